# baseline (speedup 1.0000x reference)
import functools

import jax
import jax.numpy as jnp
from jax import lax
from jax.experimental import pallas as pl
from jax.experimental.pallas import tpu as pltpu
from jax.experimental.pallas import tpu_sc as plsc

_NC = 2
_NS = 16
_NW = _NC * _NS
_L = 16

_EPS = 1e-5
_SLOPE = 0.2


def _sc_mesh():
    return plsc.VectorSubcoreMesh(
        core_axis_name="c", subcore_axis_name="s",
        num_cores=_NC, num_subcores=_NS)


@functools.lru_cache(maxsize=None)
def _make_hist(n_bins, e_per_worker):
    def body(idx_hbm, out_hbm, idx_v, cnt_v):
        wid = lax.axis_index("s") * _NC + lax.axis_index("c")
        pltpu.sync_copy(idx_hbm.at[pl.ds(wid * e_per_worker, e_per_worker)],
                        idx_v)
        zeros = jnp.zeros((_L,), jnp.float32)

        def zbody(i, _):
            cnt_v[pl.ds(i * _L, _L)] = zeros
            return 0

        lax.fori_loop(0, n_bins // _L, zbody, 0, unroll=8)
        ones = jnp.ones((_L,), jnp.float32)

        def sbody(i, _):
            iv = idx_v[pl.ds(i * _L, _L)]
            plsc.addupdate_scatter(cnt_v, [iv], ones)
            return 0

        lax.fori_loop(0, e_per_worker // _L, sbody, 0, unroll=5)
        pltpu.sync_copy(cnt_v, out_hbm.at[wid])

    return pl.kernel(
        body,
        out_type=jax.ShapeDtypeStruct((_NW, n_bins), jnp.float32),
        mesh=_sc_mesh(),
        scratch_types=[
            pltpu.VMEM((e_per_worker,), jnp.int32),
            pltpu.VMEM((n_bins,), jnp.float32),
        ],
        compiler_params=pltpu.CompilerParams(needs_layout_passes=False),
    )


def _dense_body(x_ref, cnt_ref, w1_ref, w2_ref, g1_ref, b1_ref, g2_ref,
                b2_ref, g_ref, *, inv_nk, n):
    cnt = cnt_ref[...].sum(axis=0, keepdims=True)[:, :n]
    z = lax.dot_general(x_ref[0], w1_ref[...], (((0,), (0,)), ((), ())),
                        preferred_element_type=jnp.float32)
    s1 = jnp.dot(cnt, z, preferred_element_type=jnp.float32)
    q1 = jnp.dot(cnt, z * z, preferred_element_type=jnp.float32)
    m1 = s1 * inv_nk
    v1 = q1 * inv_nk - m1 * m1
    a1 = g1_ref[...] * lax.rsqrt(v1 + _EPS)
    c1 = b1_ref[...] - m1 * a1
    u = a1 * z + c1
    u = jnp.where(u >= 0, u, _SLOPE * u)
    v = jnp.dot(u, w2_ref[...], preferred_element_type=jnp.float32)
    s2 = jnp.dot(cnt, v, preferred_element_type=jnp.float32)
    q2 = jnp.dot(cnt, v * v, preferred_element_type=jnp.float32)
    m2 = s2 * inv_nk
    v2 = q2 * inv_nk - m2 * m2
    a2 = g2_ref[...] * lax.rsqrt(v2 + _EPS)
    c2 = b2_ref[...] - m2 * a2
    g = a2 * v + c2
    g_ref[pl.ds(0, n), :] = jnp.where(g >= 0, g, _SLOPE * g).astype(
        jnp.bfloat16)
    pad = g_ref.shape[0] - n
    if pad:
        g_ref[pl.ds(n, pad), :] = jnp.zeros((pad, g.shape[1]), jnp.bfloat16)


def _dense(x, counts, w1t, w2t, g1, b1, g2, b2, nk, n_tab):
    n = x.shape[2]
    o = w2t.shape[1]
    return pl.pallas_call(
        functools.partial(_dense_body, inv_nk=1.0 / float(nk), n=n),
        out_shape=jax.ShapeDtypeStruct((n_tab, o), jnp.bfloat16),
    )(x, counts, w1t, w2t, g1, b1, g2, b2)


@functools.lru_cache(maxsize=None)
def _make_gmax(n_rows, n_chunks, k, d_out):
    rows_per_chunk = 128
    dests_per_chunk = rows_per_chunk // k
    dests_per_worker = n_chunks * dests_per_chunk
    d_w = d_out // 2
    nvec = d_w // _L
    stripe = n_rows // _NS

    def body(g_hbm, idx_hbm, out_hbm, idx_v, rows_a, out_v, stage_v,
             g_sh, sem_a):
        sid = lax.axis_index("s")
        wid = sid * _NC + lax.axis_index("c")
        pltpu.sync_copy(g_hbm.at[pl.ds(sid * stripe, stripe)], stage_v)
        pltpu.sync_copy(stage_v, g_sh.at[pl.ds(sid * stripe, stripe)])
        pltpu.sync_copy(idx_hbm.at[wid], idx_v)
        plsc.subcore_barrier()

        def compute(c, rows_v):
            def jbody(j, accs):
                out = []
                for d in range(dests_per_chunk):
                    for l in range(nvec):
                        val = plsc.bitcast(
                            rows_v[d * k + j, pl.ds(l * _L, _L)],
                            jnp.bfloat16)
                        out.append(jnp.maximum(accs[d * nvec + l], val))
                return tuple(out)

            init = tuple(plsc.bitcast(rows_v[d * k, pl.ds(l * _L, _L)],
                                      jnp.bfloat16)
                         for d in range(dests_per_chunk)
                         for l in range(nvec))
            accs = init
            for j in range(1, k):
                accs = jbody(j, accs)
            for d in range(dests_per_chunk):
                for l in range(nvec):
                    out_v[c * dests_per_chunk + d, pl.ds(l * _L, _L)] = (
                        plsc.bitcast(accs[d * nvec + l], jnp.float32))

        def chunk(c, _):
            pltpu.async_copy(g_sh.at[idx_v.at[c]], rows_a, sem_a).wait()
            compute(c, rows_a)
            return 0

        lax.fori_loop(0, n_chunks, chunk, 0)
        pltpu.sync_copy(out_v, out_hbm.at[wid])

    return pl.kernel(
        body,
        out_type=jax.ShapeDtypeStruct((_NW, dests_per_worker, d_w),
                                      jnp.float32),
        mesh=_sc_mesh(),
        scratch_types=[
            pltpu.VMEM((n_chunks, rows_per_chunk), jnp.int32),
            pltpu.VMEM((rows_per_chunk, d_w), jnp.float32),
            pltpu.VMEM((dests_per_worker, d_w), jnp.float32),
            pltpu.VMEM((stripe, d_w), jnp.float32),
            pltpu.VMEM_SHARED((n_rows, d_w), jnp.float32),
            pltpu.SemaphoreType.DMA,
        ],
        compiler_params=pltpu.CompilerParams(needs_layout_passes=False,
                                             use_tc_tiling_on_sc=False),
    )


def kernel(x, idx, k, W1, gamma1, beta1, W2, gamma2, beta2):
    B, C, N = x.shape
    K = idx.shape[2]
    H = W1.shape[0]
    O = W2.shape[0]

    off = jnp.asarray(k - K, dtype=idx.dtype)
    idx_flat = idx.reshape(B * N * K) + off

    n_tab = -(-N // 256) * 256

    counts = _make_hist(n_tab, (N * K) // _NW)(idx_flat)

    G = _dense(x, counts, jnp.transpose(W1), jnp.transpose(W2),
               gamma1.reshape(1, H), beta1.reshape(1, H),
               gamma2.reshape(1, O), beta2.reshape(1, O), N * K, n_tab)

    rows_per_chunk = 128
    dests_per_chunk = rows_per_chunk // K
    dpw = -(-N // (_NW * dests_per_chunk)) * dests_per_chunk
    n_pad = _NW * dpw
    pad = jnp.zeros(((n_pad - N) * K,), dtype=idx.dtype)
    idx_g = jnp.concatenate([idx_flat, pad]).reshape(
        _NW, (dpw * K) // rows_per_chunk, rows_per_chunk)

    Gp = lax.bitcast_convert_type(G.reshape(n_tab, O // 2, 2), jnp.float32)
    out_w = _make_gmax(n_tab, (dpw * K) // rows_per_chunk, K, O)(Gp, idx_g)
    out_bf = lax.bitcast_convert_type(out_w, jnp.bfloat16)
    out = (out_bf.reshape(n_pad, O)[:N].astype(jnp.float32)
           .T.reshape(1, O, N))
    return (out, idx)


# speedup vs baseline: 1.1368x; 1.1368x over previous
import functools

import jax
import jax.numpy as jnp
from jax import lax
from jax.experimental import pallas as pl
from jax.experimental.pallas import tpu as pltpu
from jax.experimental.pallas import tpu_sc as plsc

_NC = 2
_NS = 16
_NW = _NC * _NS
_L = 16

_EPS = 1e-5
_SLOPE = 0.2


def _sc_mesh():
    return plsc.VectorSubcoreMesh(
        core_axis_name="c", subcore_axis_name="s",
        num_cores=_NC, num_subcores=_NS)


@functools.lru_cache(maxsize=None)
def _make_hist(n_bins, e_per_worker):
    def body(idx_hbm, out_hbm, idx_v, cnt_v):
        wid = lax.axis_index("s") * _NC + lax.axis_index("c")
        pltpu.sync_copy(idx_hbm.at[pl.ds(wid * e_per_worker, e_per_worker)],
                        idx_v)
        zeros = jnp.zeros((_L,), jnp.float32)

        def zbody(i, _):
            cnt_v[pl.ds(i * _L, _L)] = zeros
            return 0

        lax.fori_loop(0, n_bins // _L, zbody, 0, unroll=8)
        ones = jnp.ones((_L,), jnp.float32)

        def sbody(i, _):
            iv = idx_v[pl.ds(i * _L, _L)]
            plsc.addupdate_scatter(cnt_v, [iv], ones)
            return 0

        lax.fori_loop(0, e_per_worker // _L, sbody, 0, unroll=5)
        pltpu.sync_copy(cnt_v, out_hbm.at[wid])

    return pl.kernel(
        body,
        out_type=jax.ShapeDtypeStruct((_NW, n_bins), jnp.float32),
        mesh=_sc_mesh(),
        scratch_types=[
            pltpu.VMEM((e_per_worker,), jnp.int32),
            pltpu.VMEM((n_bins,), jnp.float32),
        ],
        compiler_params=pltpu.CompilerParams(needs_layout_passes=False),
    )


def _dense_body(x_ref, cnt_ref, w1_ref, w2_ref, g1_ref, b1_ref, g2_ref,
                b2_ref, g_ref, *, inv_nk, n):
    cnt = cnt_ref[...].sum(axis=0, keepdims=True)[:, :n]
    z = lax.dot_general(x_ref[0], w1_ref[...], (((0,), (0,)), ((), ())),
                        preferred_element_type=jnp.float32)
    s1 = jnp.dot(cnt, z, preferred_element_type=jnp.float32)
    q1 = jnp.dot(cnt, z * z, preferred_element_type=jnp.float32)
    m1 = s1 * inv_nk
    v1 = q1 * inv_nk - m1 * m1
    a1 = g1_ref[...] * lax.rsqrt(v1 + _EPS)
    c1 = b1_ref[...] - m1 * a1
    u = a1 * z + c1
    u = jnp.where(u >= 0, u, _SLOPE * u)
    v = jnp.dot(u, w2_ref[...], preferred_element_type=jnp.float32)
    s2 = jnp.dot(cnt, v, preferred_element_type=jnp.float32)
    q2 = jnp.dot(cnt, v * v, preferred_element_type=jnp.float32)
    m2 = s2 * inv_nk
    v2 = q2 * inv_nk - m2 * m2
    a2 = g2_ref[...] * lax.rsqrt(v2 + _EPS)
    c2 = b2_ref[...] - m2 * a2
    g = a2 * v + c2
    g_ref[pl.ds(0, n), :] = jnp.where(g >= 0, g, _SLOPE * g).astype(
        jnp.bfloat16)
    pad = g_ref.shape[0] - n
    if pad:
        g_ref[pl.ds(n, pad), :] = jnp.zeros((pad, g.shape[1]), jnp.bfloat16)


def _dense(x, counts, w1t, w2t, g1, b1, g2, b2, nk, n_tab):
    n = x.shape[2]
    o = w2t.shape[1]
    return pl.pallas_call(
        functools.partial(_dense_body, inv_nk=1.0 / float(nk), n=n),
        out_shape=jax.ShapeDtypeStruct((n_tab, o), jnp.bfloat16),
    )(x, counts, w1t, w2t, g1, b1, g2, b2)


@functools.lru_cache(maxsize=None)
def _make_gmax(n_rows, n_chunks, k, d_out):
    rows_per_chunk = 128
    dests_per_chunk = rows_per_chunk // k
    dests_per_worker = n_chunks * dests_per_chunk
    d_w = d_out // 2
    nvec = d_w // _L
    stripe = n_rows // _NS

    def body(g_hbm, idx_hbm, out_hbm, idx_v, rows_a, out_v, stage_v,
             g_sh, sem_a):
        sid = lax.axis_index("s")
        wid = sid * _NC + lax.axis_index("c")
        pltpu.sync_copy(g_hbm.at[pl.ds(sid * stripe, stripe)], stage_v)
        pltpu.sync_copy(stage_v, g_sh.at[pl.ds(sid * stripe, stripe)])
        pltpu.sync_copy(idx_hbm.at[wid], idx_v)
        plsc.subcore_barrier()

        def compute(c, rows_v):
            def jbody(j, accs):
                out = []
                for d in range(dests_per_chunk):
                    for l in range(nvec):
                        val = plsc.bitcast(
                            rows_v[d * k + j, pl.ds(l * _L, _L)],
                            jnp.bfloat16)
                        out.append(jnp.maximum(accs[d * nvec + l], val))
                return tuple(out)

            init = tuple(plsc.bitcast(rows_v[d * k, pl.ds(l * _L, _L)],
                                      jnp.bfloat16)
                         for d in range(dests_per_chunk)
                         for l in range(nvec))
            accs = lax.fori_loop(1, k, jbody, init)
            for d in range(dests_per_chunk):
                for l in range(nvec):
                    out_v[c * dests_per_chunk + d, pl.ds(l * _L, _L)] = (
                        plsc.bitcast(accs[d * nvec + l], jnp.float32))

        def chunk(c, _):
            pltpu.async_copy(g_sh.at[idx_v.at[c]], rows_a, sem_a).wait()
            compute(c, rows_a)
            return 0

        lax.fori_loop(0, n_chunks, chunk, 0)
        pltpu.sync_copy(out_v, out_hbm.at[wid])

    return pl.kernel(
        body,
        out_type=jax.ShapeDtypeStruct((_NW, dests_per_worker, d_w),
                                      jnp.float32),
        mesh=_sc_mesh(),
        scratch_types=[
            pltpu.VMEM((n_chunks, rows_per_chunk), jnp.int32),
            pltpu.VMEM((rows_per_chunk, d_w), jnp.float32),
            pltpu.VMEM((dests_per_worker, d_w), jnp.float32),
            pltpu.VMEM((stripe, d_w), jnp.float32),
            pltpu.VMEM_SHARED((n_rows, d_w), jnp.float32),
            pltpu.SemaphoreType.DMA,
        ],
        compiler_params=pltpu.CompilerParams(needs_layout_passes=False,
                                             use_tc_tiling_on_sc=False),
    )


def kernel(x, idx, k, W1, gamma1, beta1, W2, gamma2, beta2):
    B, C, N = x.shape
    K = idx.shape[2]
    H = W1.shape[0]
    O = W2.shape[0]

    off = jnp.asarray(k - K, dtype=idx.dtype)
    idx_flat = idx.reshape(B * N * K) + off

    n_tab = -(-N // 256) * 256

    counts = _make_hist(n_tab, (N * K) // _NW)(idx_flat)

    G = _dense(x, counts, jnp.transpose(W1), jnp.transpose(W2),
               gamma1.reshape(1, H), beta1.reshape(1, H),
               gamma2.reshape(1, O), beta2.reshape(1, O), N * K, n_tab)

    rows_per_chunk = 128
    dests_per_chunk = rows_per_chunk // K
    dpw = -(-N // (_NW * dests_per_chunk)) * dests_per_chunk
    n_pad = _NW * dpw
    pad = jnp.zeros(((n_pad - N) * K,), dtype=idx.dtype)
    idx_g = jnp.concatenate([idx_flat, pad]).reshape(
        _NW, (dpw * K) // rows_per_chunk, rows_per_chunk)

    Gp = lax.bitcast_convert_type(G.reshape(n_tab, O // 2, 2), jnp.float32)
    out_w = _make_gmax(n_tab, (dpw * K) // rows_per_chunk, K, O)(Gp, idx_g)
    out_bf = lax.bitcast_convert_type(out_w, jnp.bfloat16)
    out = (out_bf.reshape(n_pad, O)[:N].astype(jnp.float32)
           .T.reshape(1, O, N))
    return (out, idx)


# speedup vs baseline: 1.1380x; 1.0010x over previous
import functools

import jax
import jax.numpy as jnp
from jax import lax
from jax.experimental import pallas as pl
from jax.experimental.pallas import tpu as pltpu
from jax.experimental.pallas import tpu_sc as plsc

_NC = 2
_NS = 16
_NW = _NC * _NS
_L = 16

_EPS = 1e-5
_SLOPE = 0.2


def _sc_mesh():
    return plsc.VectorSubcoreMesh(
        core_axis_name="c", subcore_axis_name="s",
        num_cores=_NC, num_subcores=_NS)


@functools.lru_cache(maxsize=None)
def _make_hist(n_bins, e_per_worker):
    def body(idx_hbm, out_hbm, idx_v, cnt_v):
        wid = lax.axis_index("s") * _NC + lax.axis_index("c")
        pltpu.sync_copy(idx_hbm.at[pl.ds(wid * e_per_worker, e_per_worker)],
                        idx_v)
        zeros = jnp.zeros((_L,), jnp.float32)

        def zbody(i, _):
            cnt_v[pl.ds(i * _L, _L)] = zeros
            return 0

        lax.fori_loop(0, n_bins // _L, zbody, 0, unroll=8)
        ones = jnp.ones((_L,), jnp.float32)

        def sbody(i, _):
            iv = idx_v[pl.ds(i * _L, _L)]
            plsc.addupdate_scatter(cnt_v, [iv], ones)
            return 0

        lax.fori_loop(0, e_per_worker // _L, sbody, 0, unroll=5)
        pltpu.sync_copy(cnt_v, out_hbm.at[wid])

    return pl.kernel(
        body,
        out_type=jax.ShapeDtypeStruct((_NW, n_bins), jnp.float32),
        mesh=_sc_mesh(),
        scratch_types=[
            pltpu.VMEM((e_per_worker,), jnp.int32),
            pltpu.VMEM((n_bins,), jnp.float32),
        ],
        compiler_params=pltpu.CompilerParams(needs_layout_passes=False),
    )


def _dense_body(x_ref, cnt_ref, w1_ref, w2_ref, g1_ref, b1_ref, g2_ref,
                b2_ref, g_ref, *, inv_nk, n):
    cnt = cnt_ref[...].sum(axis=0, keepdims=True)[:, :n]
    z = lax.dot_general(x_ref[0], w1_ref[...], (((0,), (0,)), ((), ())),
                        preferred_element_type=jnp.float32)
    s1 = jnp.dot(cnt, z, preferred_element_type=jnp.float32)
    q1 = jnp.dot(cnt, z * z, preferred_element_type=jnp.float32)
    m1 = s1 * inv_nk
    v1 = q1 * inv_nk - m1 * m1
    a1 = g1_ref[...] * lax.rsqrt(v1 + _EPS)
    c1 = b1_ref[...] - m1 * a1
    u = a1 * z + c1
    u = jnp.where(u >= 0, u, _SLOPE * u)
    v = jnp.dot(u, w2_ref[...], preferred_element_type=jnp.float32)
    s2 = jnp.dot(cnt, v, preferred_element_type=jnp.float32)
    q2 = jnp.dot(cnt, v * v, preferred_element_type=jnp.float32)
    m2 = s2 * inv_nk
    v2 = q2 * inv_nk - m2 * m2
    a2 = g2_ref[...] * lax.rsqrt(v2 + _EPS)
    c2 = b2_ref[...] - m2 * a2
    g = a2 * v + c2
    g_ref[pl.ds(0, n), :] = jnp.where(g >= 0, g, _SLOPE * g).astype(
        jnp.bfloat16)
    pad = g_ref.shape[0] - n
    if pad:
        g_ref[pl.ds(n, pad), :] = jnp.zeros((pad, g.shape[1]), jnp.bfloat16)


def _dense(x, counts, w1t, w2t, g1, b1, g2, b2, nk, n_tab):
    n = x.shape[2]
    o = w2t.shape[1]
    return pl.pallas_call(
        functools.partial(_dense_body, inv_nk=1.0 / float(nk), n=n),
        out_shape=jax.ShapeDtypeStruct((n_tab, o), jnp.bfloat16),
    )(x, counts, w1t, w2t, g1, b1, g2, b2)


@functools.lru_cache(maxsize=None)
def _make_gmax(n_rows, n_chunks, k, d_out):
    rows_per_chunk = 128
    dests_per_chunk = rows_per_chunk // k
    dests_per_worker = n_chunks * dests_per_chunk
    d_w = d_out // 2
    nvec = d_w // _L
    stripe = n_rows // _NS

    def body(g_hbm, idx_hbm, out_hbm, idx_v, rows_a, out_v, stage_v,
             g_sh, sem_a):
        sid = lax.axis_index("s")
        wid = sid * _NC + lax.axis_index("c")
        pltpu.sync_copy(g_hbm.at[pl.ds(sid * stripe, stripe)], stage_v)
        pltpu.sync_copy(stage_v, g_sh.at[pl.ds(sid * stripe, stripe)])
        pltpu.sync_copy(idx_hbm.at[wid], idx_v)
        plsc.subcore_barrier()

        def compute(c, rows_v):
            def jbody(j, accs):
                out = []
                for d in range(dests_per_chunk):
                    for l in range(nvec):
                        val = plsc.bitcast(
                            rows_v[d * k + j, pl.ds(l * _L, _L)],
                            jnp.bfloat16)
                        out.append(jnp.maximum(accs[d * nvec + l], val))
                return tuple(out)

            init = tuple(plsc.bitcast(rows_v[d * k, pl.ds(l * _L, _L)],
                                      jnp.bfloat16)
                         for d in range(dests_per_chunk)
                         for l in range(nvec))
            accs = lax.fori_loop(1, k, jbody, init, unroll=4)
            for d in range(dests_per_chunk):
                for l in range(nvec):
                    out_v[c * dests_per_chunk + d, pl.ds(l * _L, _L)] = (
                        plsc.bitcast(accs[d * nvec + l], jnp.float32))

        def chunk(c, _):
            pltpu.async_copy(g_sh.at[idx_v.at[c]], rows_a, sem_a).wait()
            compute(c, rows_a)
            return 0

        lax.fori_loop(0, n_chunks, chunk, 0)
        pltpu.sync_copy(out_v, out_hbm.at[wid])

    return pl.kernel(
        body,
        out_type=jax.ShapeDtypeStruct((_NW, dests_per_worker, d_w),
                                      jnp.float32),
        mesh=_sc_mesh(),
        scratch_types=[
            pltpu.VMEM((n_chunks, rows_per_chunk), jnp.int32),
            pltpu.VMEM((rows_per_chunk, d_w), jnp.float32),
            pltpu.VMEM((dests_per_worker, d_w), jnp.float32),
            pltpu.VMEM((stripe, d_w), jnp.float32),
            pltpu.VMEM_SHARED((n_rows, d_w), jnp.float32),
            pltpu.SemaphoreType.DMA,
        ],
        compiler_params=pltpu.CompilerParams(needs_layout_passes=False,
                                             use_tc_tiling_on_sc=False),
    )


def kernel(x, idx, k, W1, gamma1, beta1, W2, gamma2, beta2):
    B, C, N = x.shape
    K = idx.shape[2]
    H = W1.shape[0]
    O = W2.shape[0]

    off = jnp.asarray(k - K, dtype=idx.dtype)
    idx_flat = idx.reshape(B * N * K) + off

    n_tab = -(-N // 256) * 256

    counts = _make_hist(n_tab, (N * K) // _NW)(idx_flat)

    G = _dense(x, counts, jnp.transpose(W1), jnp.transpose(W2),
               gamma1.reshape(1, H), beta1.reshape(1, H),
               gamma2.reshape(1, O), beta2.reshape(1, O), N * K, n_tab)

    rows_per_chunk = 128
    dests_per_chunk = rows_per_chunk // K
    dpw = -(-N // (_NW * dests_per_chunk)) * dests_per_chunk
    n_pad = _NW * dpw
    pad = jnp.zeros(((n_pad - N) * K,), dtype=idx.dtype)
    idx_g = jnp.concatenate([idx_flat, pad]).reshape(
        _NW, (dpw * K) // rows_per_chunk, rows_per_chunk)

    Gp = lax.bitcast_convert_type(G.reshape(n_tab, O // 2, 2), jnp.float32)
    out_w = _make_gmax(n_tab, (dpw * K) // rows_per_chunk, K, O)(Gp, idx_g)
    out_bf = lax.bitcast_convert_type(out_w, jnp.bfloat16)
    out = (out_bf.reshape(n_pad, O)[:N].astype(jnp.float32)
           .T.reshape(1, O, N))
    return (out, idx)
